# baseline (speedup 1.0000x reference)
import functools

import jax
import jax.numpy as jnp
from jax import lax
from jax.experimental import pallas as pl
from jax.experimental.pallas import tpu as pltpu
from jax.experimental.pallas import tpu_sc as plsc

HID = 64
LANES = 16
NC, NS = 2, 16
NW = NC * NS
CHUNK = 128


def _tables_body(x_ref, y_ref, w_ref, b_ref, xp_ref, yp_ref, sp_ref):
    blk = x_ref.shape[0]
    wx = w_ref[:, :HID]
    wy = w_ref[:, HID:2 * HID]
    ws = w_ref[:, 2 * HID:2 * HID + 1]
    dims = (((1,), (1,)), ((), ()))
    xp_ref[...] = (
        lax.dot_general(x_ref[...], wx, dims, preferred_element_type=jnp.float32)
        + b_ref[...]
    )
    yp_ref[...] = lax.dot_general(
        y_ref[...], wy, dims, preferred_element_type=jnp.float32
    )
    rows = (
        lax.broadcasted_iota(jnp.int32, (blk, 1), 0) + pl.program_id(0) * blk
    ).astype(jnp.float32)
    sp_ref[...] = lax.dot_general(rows, ws, dims, preferred_element_type=jnp.float32)


def _project_tables(x_table, y_table, W, b):
    V = x_table.shape[0]
    blk = 2000
    assert V % blk == 0
    spec = pl.BlockSpec((blk, HID), lambda i: (i, 0))
    return pl.pallas_call(
        _tables_body,
        grid=(V // blk,),
        in_specs=[
            spec,
            spec,
            pl.BlockSpec((HID, 2 * HID + 1), lambda i: (0, 0)),
            pl.BlockSpec((1, HID), lambda i: (0, 0)),
        ],
        out_specs=[spec, spec, spec],
        out_shape=[jax.ShapeDtypeStruct((V, HID), jnp.float32)] * 3,
    )(x_table, y_table, W, b.reshape(1, HID))


SUB = 128
KSUB = CHUNK // SUB


def _make_sc_gather(N):
    rows_per_w = N // NW
    nchunk = rows_per_w // CHUNK
    nblk_w = rows_per_w // SUB
    assert rows_per_w % CHUNK == 0 and nchunk % 2 == 0 and nchunk >= 4

    mesh = plsc.VectorSubcoreMesh(core_axis_name="c", subcore_axis_name="s")

    idx_t = pltpu.VMEM((KSUB, SUB), jnp.int32)
    buf_t = pltpu.VMEM((CHUNK, HID), jnp.float32)
    stg_t = pltpu.VMEM((CHUNK * HID // 128, 128), jnp.float32)
    owid = CHUNK * HID // 128

    @functools.partial(
        pl.kernel,
        out_type=jax.ShapeDtypeStruct((N * HID // 128, 128), jnp.float32),
        mesh=mesh,
        scratch_types=[idx_t] * 6 + [buf_t] * 6 + [stg_t] * 2
        + [pltpu.SemaphoreType.DMA] * 6,
        compiler_params=pltpu.CompilerParams(use_tc_tiling_on_sc=False),
    )
    def sc_gather(idx_hbm, xp_hbm, yp_hbm, sp_hbm, out_hbm, *scratch):
        idxs0, idxs1 = scratch[0:3], scratch[3:6]
        bufs0, bufs1 = scratch[6:9], scratch[9:12]
        stg0, stg1 = scratch[12:14]
        isem0, isem1, gsem0, gsem1, wsem0, wsem1 = scratch[14:20]
        sets = (
            (idxs0, bufs0, stg0, isem0, gsem0, wsem0),
            (idxs1, bufs1, stg1, isem1, gsem1, wsem1),
        )
        tables = (xp_hbm, yp_hbm, sp_hbm)

        wid = lax.axis_index("s") * NC + lax.axis_index("c")
        row_base = wid * rows_per_w
        blk_base = wid * nblk_w

        def issue_idx(s, c):
            idx = s[0]
            isem = s[3]
            blk = blk_base + c * KSUB
            for t in range(3):
                pltpu.async_copy(idx_hbm.at[t, pl.ds(blk, KSUB)], idx[t], isem)

        def wait_idx(s):
            idx = s[0]
            isem = s[3]
            for t in range(3):
                pltpu.make_async_copy(
                    idx_hbm.at[t, pl.ds(0, KSUB)], idx[t], isem
                ).wait()

        def issue_gather(s, c):
            idx, buf = s[0], s[1]
            gsem = s[4]
            for t in range(3):
                for j in range(KSUB):
                    pltpu.async_copy(
                        tables[t].at[idx[t].at[j]],
                        buf[t].at[pl.ds(j * SUB, SUB)],
                        gsem,
                    )

        def wait_gather(s):
            buf = s[1]
            gsem = s[4]
            for t in range(3):
                pltpu.make_async_copy(
                    tables[t].at[pl.ds(0, CHUNK)], buf[t], gsem
                ).wait()

        def issue_write(s, c):
            stg = s[2]
            wsem = s[5]
            off = (row_base + c * CHUNK) * HID // 128
            pltpu.async_copy(stg, out_hbm.at[pl.ds(off, owid)], wsem)

        def wait_write(s):
            stg = s[2]
            wsem = s[5]
            pltpu.make_async_copy(stg, out_hbm.at[pl.ds(0, owid)], wsem).wait()

        def combine(s):
            (bx, by, bs), stg = s[1], s[2]

            def row_body(i2, c):
                for h in range(2):
                    r = i2 * 2 + h
                    for j in range(HID // LANES):
                        sl = pl.ds(j * LANES, LANES)
                        dsl = pl.ds(h * HID + j * LANES, LANES)
                        stg[i2, dsl] = bx[r, sl] + by[r, sl] + bs[r, sl]
                return c

            lax.fori_loop(0, CHUNK // 2, row_body, 0, unroll=2)

        issue_idx(sets[0], 0)
        issue_idx(sets[1], 1)
        wait_idx(sets[0])
        issue_gather(sets[0], 0)

        def outer(i, carry):
            g = i * 2
            for b in range(2):
                s = sets[b]
                so = sets[1 - b]
                c = g + b
                wait_gather(s)

                @pl.when(c + 2 < nchunk)
                def _():
                    issue_idx(s, c + 2)

                @pl.when(c + 1 < nchunk)
                def _():
                    wait_idx(so)

                    @pl.when(c >= 1)
                    def _():
                        wait_write(so)

                    issue_gather(so, c + 1)

                combine(s)
                issue_write(s, c)
            return carry

        lax.fori_loop(0, nchunk // 2, outer, 0)
        wait_write(sets[0])
        wait_write(sets[1])

    return sc_gather


EPI_BB = 8


def _epilogue(out128, B, L):
    rows_per_bb = EPI_BB * L * HID // 128

    def body(in_ref, out_ref):
        a = in_ref[...]
        left = a[:, :HID]
        right = a[:, HID:]
        inter = jnp.stack((left, right), axis=1)
        out_ref[...] = inter.reshape(EPI_BB, L, HID)

    return pl.pallas_call(
        body,
        grid=(B // EPI_BB,),
        in_specs=[pl.BlockSpec((rows_per_bb, 128), lambda i: (i, 0))],
        out_specs=pl.BlockSpec((EPI_BB, L, HID), lambda i: (i, 0, 0)),
        out_shape=jax.ShapeDtypeStruct((B, L, HID), jnp.float32),
    )(out128)


def kernel(src, x_table, y_table, W, b):
    B, L, _ = src.shape
    N = B * L
    idx = jnp.transpose(src, (2, 0, 1)).astype(jnp.int32).reshape(3, N // SUB, SUB)
    xp, yp, sp = _project_tables(x_table, y_table, W, b)
    out = _make_sc_gather(N)(idx, xp, yp, sp)
    return _epilogue(out, B, L)


# speedup vs baseline: 1.4035x; 1.4035x over previous
import functools

import jax
import jax.numpy as jnp
from jax import lax
from jax.experimental import pallas as pl
from jax.experimental.pallas import tpu as pltpu
from jax.experimental.pallas import tpu_sc as plsc

HID = 64
LANES = 16
NC, NS = 2, 16
NW = NC * NS
CHUNK = 128


BLK = 2000


def _tables_body(x_ref, y_ref, w_ref, b_ref, txy_ref, ts_ref):
    wx = w_ref[:, :HID]
    wy = w_ref[:, HID:2 * HID]
    ws = w_ref[:, 2 * HID:2 * HID + 1]
    dims = (((1,), (1,)), ((), ()))
    xp = (
        lax.dot_general(x_ref[...], wx, dims, preferred_element_type=jnp.float32)
        + b_ref[...]
    )
    yp = lax.dot_general(y_ref[...], wy, dims, preferred_element_type=jnp.float32)
    txy_ref[...] = jnp.concatenate([xp, yp], axis=1)
    rows = (
        lax.broadcasted_iota(jnp.int32, (BLK, 1), 0) + pl.program_id(0) * BLK
    ).astype(jnp.float32)
    dims = (((1,), (1,)), ((), ()))
    sp = lax.dot_general(rows, ws, dims, preferred_element_type=jnp.float32)
    ts_ref[...] = jnp.concatenate([sp, sp], axis=1)


def _project_tables(x_table, y_table, W, b):
    V = x_table.shape[0]
    assert V % BLK == 0
    tspec = pl.BlockSpec((BLK, HID), lambda i: (i, 0))
    ospec = pl.BlockSpec((BLK, 2 * HID), lambda i: (i, 0))
    return pl.pallas_call(
        _tables_body,
        grid=(V // BLK,),
        in_specs=[
            tspec,
            tspec,
            pl.BlockSpec((HID, 2 * HID + 1), lambda i: (0, 0)),
            pl.BlockSpec((1, HID), lambda i: (0, 0)),
        ],
        out_specs=[ospec, ospec],
        out_shape=[jax.ShapeDtypeStruct((V, 2 * HID), jnp.float32)] * 2,
    )(x_table, y_table, W, b.reshape(1, HID))


SUB = 128
KSUB = CHUNK // SUB


def _make_sc_gather(N):
    rows_per_w = N // NW
    nchunk = rows_per_w // CHUNK
    nblk_w = rows_per_w // SUB
    assert rows_per_w % CHUNK == 0 and nchunk % 2 == 0 and nchunk >= 4

    mesh = plsc.VectorSubcoreMesh(core_axis_name="c", subcore_axis_name="s")

    idx_t = pltpu.VMEM((KSUB, SUB), jnp.int32)
    buf_t = pltpu.VMEM((CHUNK, HID), jnp.float32)

    @functools.partial(
        pl.kernel,
        out_type=jax.ShapeDtypeStruct((N, HID), jnp.float32),
        mesh=mesh,
        scratch_types=[idx_t] * 6 + [buf_t] * 6 + [pltpu.SemaphoreType.DMA] * 6,
        compiler_params=pltpu.CompilerParams(use_tc_tiling_on_sc=False),
    )
    def sc_gather(idx_hbm, txy_hbm, ts_hbm, out_hbm, *scratch):
        idxs0, idxs1 = scratch[0:3], scratch[3:6]
        bufs0, bufs1 = scratch[6:9], scratch[9:12]
        isem0, isem1, gsem0, gsem1, wsem0, wsem1 = scratch[12:18]
        sets = (
            (idxs0, bufs0, isem0, gsem0, wsem0),
            (idxs1, bufs1, isem1, gsem1, wsem1),
        )
        tables = (txy_hbm, txy_hbm, ts_hbm)

        wid = lax.axis_index("s") * NC + lax.axis_index("c")
        row_base = wid * rows_per_w
        blk_base = wid * nblk_w

        def issue_idx(s, c):
            idx, _, isem, _, _ = s
            blk = blk_base + c * KSUB
            for t in range(3):
                pltpu.async_copy(idx_hbm.at[t, pl.ds(blk, KSUB)], idx[t], isem)

        def wait_idx(s):
            idx, _, isem, _, _ = s
            for t in range(3):
                pltpu.make_async_copy(
                    idx_hbm.at[t, pl.ds(0, KSUB)], idx[t], isem
                ).wait()

        def issue_gather(s, c):
            idx, buf, _, gsem, _ = s
            for t in range(3):
                for j in range(KSUB):
                    pltpu.async_copy(
                        tables[t].at[idx[t].at[j]],
                        buf[t].at[pl.ds(j * SUB, SUB)],
                        gsem,
                    )

        def wait_gather(s):
            _, buf, _, gsem, _ = s
            for t in range(3):
                pltpu.make_async_copy(
                    out_hbm.at[pl.ds(0, CHUNK)], buf[t], gsem
                ).wait()

        def issue_write(s, c):
            _, buf, _, _, wsem = s
            off = row_base + c * CHUNK
            pltpu.async_copy(buf[0], out_hbm.at[pl.ds(off, CHUNK)], wsem)

        def wait_write(s):
            _, buf, _, _, wsem = s
            pltpu.make_async_copy(
                buf[0], out_hbm.at[pl.ds(0, CHUNK)], wsem
            ).wait()

        def combine(s):
            _, buf, _, _, _ = s
            bx, by, bs = buf

            def row_body(i, c):
                for j in range(HID // LANES):
                    sl = pl.ds(j * LANES, LANES)
                    bx[i, sl] = bx[i, sl] + by[i, sl] + bs[i, sl]
                return c

            lax.fori_loop(0, CHUNK, row_body, 0, unroll=2)

        issue_idx(sets[0], 0)
        issue_idx(sets[1], 1)
        wait_idx(sets[0])
        issue_gather(sets[0], 0)

        def outer(i, carry):
            g = i * 2
            for b in range(2):
                s = sets[b]
                so = sets[1 - b]
                c = g + b
                wait_gather(s)

                @pl.when(c + 2 < nchunk)
                def _():
                    issue_idx(s, c + 2)

                @pl.when(c + 1 < nchunk)
                def _():
                    wait_idx(so)

                    @pl.when(c >= 1)
                    def _():
                        wait_write(so)

                    issue_gather(so, c + 1)

                combine(s)
                issue_write(s, c)
            return carry

        lax.fori_loop(0, nchunk // 2, outer, 0)
        wait_write(sets[0])
        wait_write(sets[1])

    return sc_gather


def kernel(src, x_table, y_table, W, b):
    B, L, _ = src.shape
    N = B * L
    V = x_table.shape[0]
    half = jnp.array([0, 1, 0], jnp.int32)[:, None, None]
    idx = (
        jnp.transpose(src, (2, 0, 1)).astype(jnp.int32) * 2 + half
    ).reshape(3, N // SUB, SUB)
    txy, ts = _project_tables(x_table, y_table, W, b)
    out = _make_sc_gather(N)(idx, txy.reshape(2 * V, HID), ts.reshape(2 * V, HID))
    return out.reshape(B, L, HID)


# speedup vs baseline: 1.4079x; 1.0031x over previous
import functools

import jax
import jax.numpy as jnp
from jax import lax
from jax.experimental import pallas as pl
from jax.experimental.pallas import tpu as pltpu
from jax.experimental.pallas import tpu_sc as plsc

HID = 64
LANES = 16
NC, NS = 2, 16
NW = NC * NS
CHUNK = 256


BLK = 2000


def _tables_body(x_ref, y_ref, w_ref, b_ref, txy_ref, ts_ref):
    wx = w_ref[:, :HID]
    wy = w_ref[:, HID:2 * HID]
    ws = w_ref[:, 2 * HID:2 * HID + 1]
    dims = (((1,), (1,)), ((), ()))
    xp = (
        lax.dot_general(x_ref[...], wx, dims, preferred_element_type=jnp.float32)
        + b_ref[...]
    )
    yp = lax.dot_general(y_ref[...], wy, dims, preferred_element_type=jnp.float32)
    txy_ref[...] = jnp.concatenate([xp, yp], axis=1)
    rows = (
        lax.broadcasted_iota(jnp.int32, (BLK, 1), 0) + pl.program_id(0) * BLK
    ).astype(jnp.float32)
    dims = (((1,), (1,)), ((), ()))
    sp = lax.dot_general(rows, ws, dims, preferred_element_type=jnp.float32)
    ts_ref[...] = jnp.concatenate([sp, sp], axis=1)


def _project_tables(x_table, y_table, W, b):
    V = x_table.shape[0]
    assert V % BLK == 0
    tspec = pl.BlockSpec((BLK, HID), lambda i: (i, 0))
    ospec = pl.BlockSpec((BLK, 2 * HID), lambda i: (i, 0))
    return pl.pallas_call(
        _tables_body,
        grid=(V // BLK,),
        in_specs=[
            tspec,
            tspec,
            pl.BlockSpec((HID, 2 * HID + 1), lambda i: (0, 0)),
            pl.BlockSpec((1, HID), lambda i: (0, 0)),
        ],
        out_specs=[ospec, ospec],
        out_shape=[jax.ShapeDtypeStruct((V, 2 * HID), jnp.float32)] * 2,
    )(x_table, y_table, W, b.reshape(1, HID))


SUB = 128
KSUB = CHUNK // SUB


def _make_sc_gather(N):
    rows_per_w = N // NW
    nchunk = rows_per_w // CHUNK
    nblk_w = rows_per_w // SUB
    assert rows_per_w % CHUNK == 0 and nchunk % 2 == 0 and nchunk >= 4

    mesh = plsc.VectorSubcoreMesh(core_axis_name="c", subcore_axis_name="s")

    idx_t = pltpu.VMEM((KSUB, SUB), jnp.int32)
    buf_t = pltpu.VMEM((CHUNK, HID), jnp.float32)

    @functools.partial(
        pl.kernel,
        out_type=jax.ShapeDtypeStruct((N, HID), jnp.float32),
        mesh=mesh,
        scratch_types=[idx_t] * 6 + [buf_t] * 6 + [pltpu.SemaphoreType.DMA] * 6,
        compiler_params=pltpu.CompilerParams(use_tc_tiling_on_sc=False),
    )
    def sc_gather(idx_hbm, txy_hbm, ts_hbm, out_hbm, *scratch):
        idxs0, idxs1 = scratch[0:3], scratch[3:6]
        bufs0, bufs1 = scratch[6:9], scratch[9:12]
        isem0, isem1, gsem0, gsem1, wsem0, wsem1 = scratch[12:18]
        sets = (
            (idxs0, bufs0, isem0, gsem0, wsem0),
            (idxs1, bufs1, isem1, gsem1, wsem1),
        )
        tables = (txy_hbm, txy_hbm, ts_hbm)

        wid = lax.axis_index("s") * NC + lax.axis_index("c")
        row_base = wid * rows_per_w
        blk_base = wid * nblk_w

        def issue_idx(s, c):
            idx, _, isem, _, _ = s
            blk = blk_base + c * KSUB
            for t in range(3):
                pltpu.async_copy(idx_hbm.at[t, pl.ds(blk, KSUB)], idx[t], isem)

        def wait_idx(s):
            idx, _, isem, _, _ = s
            for t in range(3):
                pltpu.make_async_copy(
                    idx_hbm.at[t, pl.ds(0, KSUB)], idx[t], isem
                ).wait()

        def issue_gather(s, c):
            idx, buf, _, gsem, _ = s
            for t in range(3):
                for j in range(KSUB):
                    pltpu.async_copy(
                        tables[t].at[idx[t].at[j]],
                        buf[t].at[pl.ds(j * SUB, SUB)],
                        gsem,
                    )

        def wait_gather(s):
            _, buf, _, gsem, _ = s
            for t in range(3):
                pltpu.make_async_copy(
                    out_hbm.at[pl.ds(0, CHUNK)], buf[t], gsem
                ).wait()

        def issue_write(s, c):
            _, buf, _, _, wsem = s
            off = row_base + c * CHUNK
            pltpu.async_copy(buf[0], out_hbm.at[pl.ds(off, CHUNK)], wsem)

        def wait_write(s):
            _, buf, _, _, wsem = s
            pltpu.make_async_copy(
                buf[0], out_hbm.at[pl.ds(0, CHUNK)], wsem
            ).wait()

        def combine(s):
            _, buf, _, _, _ = s
            bx, by, bs = buf

            def row_body(i, c):
                for j in range(HID // LANES):
                    sl = pl.ds(j * LANES, LANES)
                    bx[i, sl] = bx[i, sl] + by[i, sl] + bs[i, sl]
                return c

            lax.fori_loop(0, CHUNK, row_body, 0, unroll=2)

        issue_idx(sets[0], 0)
        issue_idx(sets[1], 1)
        wait_idx(sets[0])
        issue_gather(sets[0], 0)

        def outer(i, carry):
            g = i * 2
            for b in range(2):
                s = sets[b]
                so = sets[1 - b]
                c = g + b
                wait_gather(s)

                @pl.when(c + 2 < nchunk)
                def _():
                    issue_idx(s, c + 2)

                @pl.when(c + 1 < nchunk)
                def _():
                    wait_idx(so)

                    @pl.when(c >= 1)
                    def _():
                        wait_write(so)

                    issue_gather(so, c + 1)

                combine(s)
                issue_write(s, c)
            return carry

        lax.fori_loop(0, nchunk // 2, outer, 0)
        wait_write(sets[0])
        wait_write(sets[1])

    return sc_gather


def kernel(src, x_table, y_table, W, b):
    B, L, _ = src.shape
    N = B * L
    V = x_table.shape[0]
    half = jnp.array([0, 1, 0], jnp.int32)[:, None, None]
    idx = (
        jnp.transpose(src, (2, 0, 1)).astype(jnp.int32) * 2 + half
    ).reshape(3, N // SUB, SUB)
    txy, ts = _project_tables(x_table, y_table, W, b)
    out = _make_sc_gather(N)(idx, txy.reshape(2 * V, HID), ts.reshape(2 * V, HID))
    return out.reshape(B, L, HID)
